# baseline (speedup 1.0000x reference)
import dataclasses
import functools

import jax
import jax.numpy as jnp
from jax import lax
from jax.experimental import pallas as pl
from jax.experimental.pallas import tpu as pltpu
from jax.experimental.pallas import tpu_sc as plsc

_N = 10000
_E = 320000
_D = 128
_K = 128
_NCHUNK = _E // _K
_NC = 2
_NS = 16
_NW = _NC * _NS
_CHB = 8
_NBATCH = -(-_NCHUNK // _CHB)
_BSTEPS = -(-_NBATCH // _NW)
_N_PAD = 10240
_RPT = _N_PAD // _NS
_ZR = 128
_EPS = 1e-5
_B = 1024

_mesh = plsc.VectorSubcoreMesh(core_axis_name="c", subcore_axis_name="s",
                               num_cores=_NC)

_sc_params = pltpu.CompilerParams()
if "needs_layout_passes" in pltpu.CompilerParams.__dataclass_fields__:
    _sc_params = dataclasses.replace(_sc_params, needs_layout_passes=False)


@functools.partial(
    pl.kernel,
    compiler_params=_sc_params,
    out_type=[jax.ShapeDtypeStruct((_N_PAD, _D), jnp.float32),
              jax.ShapeDtypeStruct((_N_PAD, _D), jnp.float32),
              jax.ShapeDtypeStruct((_NC, _N_PAD // _D, _D), jnp.float32)],
    mesh=_mesh,
    scratch_types=[
        pltpu.VMEM((_CHB, _K), jnp.int32),
        pltpu.VMEM((_CHB, _K), jnp.int32),
        pltpu.VMEM((_CHB, _K), jnp.int32),
        pltpu.VMEM((_CHB, _K), jnp.int32),
        pltpu.VMEM((_K, _D), jnp.float32),
        pltpu.VMEM((_K, _D), jnp.float32),
        pltpu.VMEM((_N_PAD // _D, _D), jnp.float32),
        pltpu.VMEM((_N_PAD // _D,), jnp.int32),
        pltpu.VMEM_SHARED((_N_PAD, _D), jnp.float32),
        pltpu.VMEM_SHARED((_N_PAD // _D, _D), jnp.float32),
        pltpu.SemaphoreType.DMA,
        pltpu.SemaphoreType.DMA,
        pltpu.SemaphoreType.DMA,
        pltpu.SemaphoreType.DMA,
        pltpu.SemaphoreType.DMA,
        pltpu.SemaphoreType.DMA,
    ],
)
def _sc_aggregate(table_hbm, src_hbm, dst_hbm, out0_hbm, out1_hbm, cnt_hbm,
                  sb0, sb1, db0, db1, r0, r1, hist, idrows,
                  acc, cnt_sh, gsem0, gsem1, isem0, isem1, ssem0, ssem1):
    c = lax.axis_index("c")
    s = lax.axis_index("s")
    w = c * _NS + s
    row0 = pl.multiple_of(s * _RPT, 8)
    ones16 = jnp.ones((16,), jnp.float32)
    _HR = _N_PAD // _D
    sb = (sb0, sb1)
    db = (db0, db1)
    rv = (r0, r1)
    gsem = (gsem0, gsem1)
    isem = (isem0, isem1)
    ssem = (ssem0, ssem1)

    def _idx_copies(batch, p):
        brow = pl.multiple_of(batch * _CHB, 8)
        return (pltpu.make_async_copy(src_hbm.at[pl.ds(brow, _CHB)], sb[p],
                                      isem[p]),
                pltpu.make_async_copy(dst_hbm.at[pl.ds(brow, _CHB)], db[p],
                                      isem[p]))

    def _prefetch_idx(batch, p):
        @pl.when(batch < _NBATCH)
        def _():
            for cp in _idx_copies(batch, p):
                cp.start()

    def _wait_idx(batch, p):
        @pl.when(batch < _NBATCH)
        def _():
            for cp in _idx_copies(batch, p):
                cp.wait()

    _prefetch_idx(w, 0)
    _prefetch_idx(w + _NW, 1)

    @pl.loop(0, _ZR)
    def _(r):
        for col in range(0, _D, 16):
            r1[r, pl.ds(col, 16)] = jnp.zeros((16,), jnp.float32)

    @pl.loop(0, _HR)
    def _(r):
        for col in range(0, _D, 16):
            hist[r, pl.ds(col, 16)] = jnp.zeros((16,), jnp.float32)

    for g in range(_HR // 16):
        idrows[pl.ds(g * 16, 16)] = (lax.iota(jnp.int32, 16)
                                     + jnp.int32(g * 16))

    for b in range(_RPT // _ZR):
        pltpu.sync_copy(r1, acc.at[pl.ds(row0 + b * _ZR, _ZR)])

    @pl.when(s == 0)
    def _():
        pltpu.sync_copy(r1.at[pl.ds(0, _HR)], cnt_sh)
    plsc.subcore_barrier()

    def _gather(batch, jj, p, wait_prev=True):
        @pl.when(batch * _CHB + jj < _NCHUNK)
        def _():
            if wait_prev:
                pltpu.make_async_copy(rv[jj % 2], acc.at[db[p].at[jj]],
                                      ssem[jj % 2]).wait()
            pltpu.async_copy(table_hbm.at[sb[p].at[jj]], rv[jj % 2],
                             gsem[jj % 2])

    def _drain(batch, jj, p):
        @pl.when(batch * _CHB + jj < _NCHUNK)
        def _():
            for g in range(_K // 16):
                dvec = db[p][jj, pl.ds(g * 16, 16)]
                plsc.addupdate_scatter(
                    hist,
                    [lax.shift_right_logical(dvec, 7),
                     lax.bitwise_and(dvec, jnp.int32(_D - 1))],
                    ones16)
            pltpu.make_async_copy(table_hbm.at[sb[p].at[jj]], rv[jj % 2],
                                  gsem[jj % 2]).wait()
            pltpu.async_copy(rv[jj % 2], acc.at[db[p].at[jj]], ssem[jj % 2],
                             add=True)

    def _half_iter(batch, p, first=False):
        nxt = batch + _NW
        for jj in range(_CHB - 1):
            _gather(batch, jj + 1, p, wait_prev=(not first) or jj + 1 >= 2)
            _drain(batch, jj, p)
            if jj == 0 and not first:
                _prefetch_idx(nxt, 1 - p)
        _wait_idx(nxt, 1 - p)
        _gather(nxt, 0, 1 - p)
        _drain(batch, _CHB - 1, p)

    _wait_idx(w, 0)
    _gather(w, 0, 0, wait_prev=False)
    _half_iter(w, 0, first=True)

    @pl.loop(1, _BSTEPS, step=2)
    def _(jb):
        _half_iter(w + jb * _NW, 1)
        _half_iter(w + (jb + 1) * _NW, 0)

    pltpu.make_async_copy(rv[0], acc.at[db[0].at[0]], ssem[0]).wait()
    pltpu.make_async_copy(rv[1], acc.at[db[0].at[1]], ssem[1]).wait()

    pltpu.sync_copy(hist, cnt_sh.at[idrows], add=True)
    plsc.subcore_barrier()

    @pl.when(s == 0)
    def _():
        pltpu.sync_copy(cnt_sh, cnt_hbm.at[c])

    @pl.when(c == 0)
    def _():
        pltpu.sync_copy(acc.at[pl.ds(row0, _RPT)],
                        out0_hbm.at[pl.ds(row0, _RPT)])

    @pl.when(c == 1)
    def _():
        pltpu.sync_copy(acc.at[pl.ds(row0, _RPT)],
                        out1_hbm.at[pl.ds(row0, _RPT)])


def _dot_t(a, w):
    return lax.dot_general(a, w, (((1,), (1,)), ((), ())),
                           preferred_element_type=jnp.float32)


def _mean_conv(pa_ref, pb_ref, ca_ref, cb_ref, h_ref, wl_ref, wr_ref, b_ref):
    ic = 1.0 / jnp.maximum(ca_ref[0] + cb_ref[0], 1.0)
    ri = lax.broadcasted_iota(jnp.int32, (_B, 1), 0)
    sel_row = (lax.broadcasted_iota(jnp.int32, (_B, 8), 1)
               == ri // _D).astype(jnp.float32)
    sel_lane = (lax.broadcasted_iota(jnp.int32, (_B, _D), 1)
                == ri % _D).astype(jnp.float32)
    t = lax.dot_general(sel_row, ic, (((1,), (0,)), ((), ())),
                        preferred_element_type=jnp.float32)
    inv_col = jnp.sum(t * sel_lane, axis=1, keepdims=True)
    mean = (pa_ref[...] + pb_ref[...]) * inv_col
    y = _dot_t(mean, wl_ref[...]) + _dot_t(h_ref[...], wr_ref[...]) \
        + b_ref[...]
    return jnp.maximum(y, 0.0)


def _layer1_body(pa_ref, pb_ref, ca_ref, cb_ref, h_ref, wl_ref, wr_ref,
                 b_ref, out_ref):
    out_ref[...] = _mean_conv(pa_ref, pb_ref, ca_ref, cb_ref, h_ref,
                              wl_ref, wr_ref, b_ref)


def _layer2fc_body(pa_ref, pb_ref, ca_ref, cb_ref, h_ref, wl_ref, wr_ref,
                   b_ref, gamma_ref, beta_ref, fcw_ref, fcb_ref,
                   out_ref, h2buf, accum):
    ph = pl.program_id(0)
    i = pl.program_id(1)

    @pl.when(ph == 0)
    def _():
        h2 = _mean_conv(pa_ref, pb_ref, ca_ref, cb_ref, h_ref, wl_ref,
                        wr_ref, b_ref)
        h2buf[pl.ds(i * _B, _B), :] = h2

        rows = i * _B + lax.broadcasted_iota(jnp.int32, (_B, 1), 0)
        h2m = jnp.where(rows < _N, h2, 0.0)

        @pl.when(i == 0)
        def _():
            accum[...] = jnp.zeros_like(accum)

        accum[0:1, :] += jnp.sum(h2m, axis=0, keepdims=True)
        accum[1:2, :] += jnp.sum(h2m * h2m, axis=0, keepdims=True)

    @pl.when(ph == 1)
    def _():
        inv_n = 1.0 / _N
        mu = accum[0:1, :] * inv_n
        var = accum[1:2, :] * inv_n - mu * mu
        a = gamma_ref[...] * lax.rsqrt(var + _EPS)
        c0 = beta_ref[...] - mu * a
        h2 = h2buf[pl.ds(i * _B, _B), :]
        out_ref[...] = _dot_t(h2 * a + c0, fcw_ref[...]) + fcb_ref[...]


def _full(shape):
    return pl.BlockSpec(shape, lambda i: tuple(0 for _ in shape))


def _sage_layer1(part_a, part_b, cnt3, h, Wl, Wr, b):
    nblk = _N_PAD // _B
    in_specs = [
        pl.BlockSpec((_B, _D), lambda i: (i, 0)),
        pl.BlockSpec((_B, _D), lambda i: (i, 0)),
        pl.BlockSpec((1, 8, _D), lambda i: (0, i, 0)),
        pl.BlockSpec((1, 8, _D), lambda i: (1, i, 0)),
        pl.BlockSpec((_B, _D), lambda i: (i, 0)),
        _full((_D, _D)),
        _full((_D, _D)),
        _full((1, _D)),
    ]
    return pl.pallas_call(
        _layer1_body,
        grid=(nblk,),
        in_specs=in_specs,
        out_specs=pl.BlockSpec((_B, _D), lambda i: (i, 0)),
        out_shape=jax.ShapeDtypeStruct((_N, _D), jnp.float32),
    )(part_a, part_b, cnt3, cnt3, h, Wl, Wr, b.reshape(1, _D))


def kernel(x, edge_index, W1l, b1, W1r, W2l, b2, W2r, gamma, beta, fcW, fcb):
    src = edge_index[0].reshape(_NCHUNK, _K)
    dst = edge_index[1].reshape(_NCHUNK, _K)

    p1a, p1b, cnt3 = _sc_aggregate(x, src, dst)
    h1 = _sage_layer1(p1a, p1b, cnt3, x, W1l, W1r, b1)

    p2a, p2b, _ = _sc_aggregate(h1, src, dst)

    nblk = _N_PAD // _B
    in_specs = [
        pl.BlockSpec((_B, _D), lambda ph, i: (i * (1 - ph), 0)),
        pl.BlockSpec((_B, _D), lambda ph, i: (i * (1 - ph), 0)),
        pl.BlockSpec((1, 8, _D), lambda ph, i: (0, i * (1 - ph), 0)),
        pl.BlockSpec((1, 8, _D), lambda ph, i: (1, i * (1 - ph), 0)),
        pl.BlockSpec((_B, _D), lambda ph, i: (i * (1 - ph), 0)),
        pl.BlockSpec((_D, _D), lambda ph, i: (0, 0)),
        pl.BlockSpec((_D, _D), lambda ph, i: (0, 0)),
        pl.BlockSpec((1, _D), lambda ph, i: (0, 0)),
        pl.BlockSpec((1, _D), lambda ph, i: (0, 0)),
        pl.BlockSpec((1, _D), lambda ph, i: (0, 0)),
        pl.BlockSpec((_D, _D), lambda ph, i: (0, 0)),
        pl.BlockSpec((1, _D), lambda ph, i: (0, 0)),
    ]
    out = pl.pallas_call(
        _layer2fc_body,
        grid=(2, nblk),
        in_specs=in_specs,
        out_specs=pl.BlockSpec((_B, _D), lambda ph, i: (i * ph, 0)),
        out_shape=jax.ShapeDtypeStruct((_N, _D), jnp.float32),
        scratch_shapes=[pltpu.VMEM((_N_PAD, _D), jnp.float32),
                        pltpu.VMEM((8, _D), jnp.float32)],
    )(p2a, p2b, cnt3, cnt3, h1, W2l, W2r, b2.reshape(1, _D),
      gamma.reshape(1, _D), beta.reshape(1, _D), fcW, fcb.reshape(1, _D))
    return out


# speedup vs baseline: 1.0255x; 1.0255x over previous
import dataclasses
import functools

import jax
import jax.numpy as jnp
from jax import lax
from jax.experimental import pallas as pl
from jax.experimental.pallas import tpu as pltpu
from jax.experimental.pallas import tpu_sc as plsc

_N = 10000
_E = 320000
_D = 128
_K = 128
_NCHUNK = _E // _K
_NC = 2
_NS = 16
_NW = _NC * _NS
_CHB = 8
_NBATCH = -(-_NCHUNK // _CHB)
_BSTEPS = -(-_NBATCH // _NW)
_N_PAD = 10240
_RPT = _N_PAD // _NS
_ZR = 128
_EPS = 1e-5
_B = 2048
_SB = _B // _D

_mesh = plsc.VectorSubcoreMesh(core_axis_name="c", subcore_axis_name="s",
                               num_cores=_NC)

_sc_params = pltpu.CompilerParams()
if "needs_layout_passes" in pltpu.CompilerParams.__dataclass_fields__:
    _sc_params = dataclasses.replace(_sc_params, needs_layout_passes=False)


@functools.partial(
    pl.kernel,
    compiler_params=_sc_params,
    out_type=[jax.ShapeDtypeStruct((_N_PAD, _D), jnp.float32),
              jax.ShapeDtypeStruct((_N_PAD, _D), jnp.float32),
              jax.ShapeDtypeStruct((_NC, _N_PAD // _D, _D), jnp.float32)],
    mesh=_mesh,
    scratch_types=[
        pltpu.VMEM((_CHB, _K), jnp.int32),
        pltpu.VMEM((_CHB, _K), jnp.int32),
        pltpu.VMEM((_CHB, _K), jnp.int32),
        pltpu.VMEM((_CHB, _K), jnp.int32),
        pltpu.VMEM((_K, _D), jnp.float32),
        pltpu.VMEM((_K, _D), jnp.float32),
        pltpu.VMEM((_N_PAD // _D, _D), jnp.float32),
        pltpu.VMEM((_N_PAD // _D,), jnp.int32),
        pltpu.VMEM_SHARED((_N_PAD, _D), jnp.float32),
        pltpu.VMEM_SHARED((_N_PAD // _D, _D), jnp.float32),
        pltpu.SemaphoreType.DMA,
        pltpu.SemaphoreType.DMA,
        pltpu.SemaphoreType.DMA,
        pltpu.SemaphoreType.DMA,
        pltpu.SemaphoreType.DMA,
        pltpu.SemaphoreType.DMA,
    ],
)
def _sc_aggregate(table_hbm, src_hbm, dst_hbm, out0_hbm, out1_hbm, cnt_hbm,
                  sb0, sb1, db0, db1, r0, r1, hist, idrows,
                  acc, cnt_sh, gsem0, gsem1, isem0, isem1, ssem0, ssem1):
    c = lax.axis_index("c")
    s = lax.axis_index("s")
    w = c * _NS + s
    row0 = pl.multiple_of(s * _RPT, 8)
    ones16 = jnp.ones((16,), jnp.float32)
    _HR = _N_PAD // _D
    sb = (sb0, sb1)
    db = (db0, db1)
    rv = (r0, r1)
    gsem = (gsem0, gsem1)
    isem = (isem0, isem1)
    ssem = (ssem0, ssem1)

    def _idx_copies(batch, p):
        brow = pl.multiple_of(batch * _CHB, 8)
        return (pltpu.make_async_copy(src_hbm.at[pl.ds(brow, _CHB)], sb[p],
                                      isem[p]),
                pltpu.make_async_copy(dst_hbm.at[pl.ds(brow, _CHB)], db[p],
                                      isem[p]))

    def _prefetch_idx(batch, p):
        @pl.when(batch < _NBATCH)
        def _():
            for cp in _idx_copies(batch, p):
                cp.start()

    def _wait_idx(batch, p):
        @pl.when(batch < _NBATCH)
        def _():
            for cp in _idx_copies(batch, p):
                cp.wait()

    _prefetch_idx(w, 0)
    _prefetch_idx(w + _NW, 1)

    @pl.loop(0, _ZR)
    def _(r):
        for col in range(0, _D, 16):
            r1[r, pl.ds(col, 16)] = jnp.zeros((16,), jnp.float32)

    @pl.loop(0, _HR)
    def _(r):
        for col in range(0, _D, 16):
            hist[r, pl.ds(col, 16)] = jnp.zeros((16,), jnp.float32)

    for g in range(_HR // 16):
        idrows[pl.ds(g * 16, 16)] = (lax.iota(jnp.int32, 16)
                                     + jnp.int32(g * 16))

    for b in range(_RPT // _ZR):
        pltpu.sync_copy(r1, acc.at[pl.ds(row0 + b * _ZR, _ZR)])

    @pl.when(s == 0)
    def _():
        pltpu.sync_copy(r1.at[pl.ds(0, _HR)], cnt_sh)
    plsc.subcore_barrier()

    def _gather(batch, jj, p, wait_prev=True):
        @pl.when(batch * _CHB + jj < _NCHUNK)
        def _():
            if wait_prev:
                pltpu.make_async_copy(rv[jj % 2], acc.at[db[p].at[jj]],
                                      ssem[jj % 2]).wait()
            pltpu.async_copy(table_hbm.at[sb[p].at[jj]], rv[jj % 2],
                             gsem[jj % 2])

    def _drain(batch, jj, p):
        @pl.when(batch * _CHB + jj < _NCHUNK)
        def _():
            for g in range(_K // 16):
                dvec = db[p][jj, pl.ds(g * 16, 16)]
                plsc.addupdate_scatter(
                    hist,
                    [lax.shift_right_logical(dvec, 7),
                     lax.bitwise_and(dvec, jnp.int32(_D - 1))],
                    ones16)
            pltpu.make_async_copy(table_hbm.at[sb[p].at[jj]], rv[jj % 2],
                                  gsem[jj % 2]).wait()
            pltpu.async_copy(rv[jj % 2], acc.at[db[p].at[jj]], ssem[jj % 2],
                             add=True)

    def _half_iter(batch, p, first=False):
        nxt = batch + _NW
        for jj in range(_CHB - 1):
            _gather(batch, jj + 1, p, wait_prev=(not first) or jj + 1 >= 2)
            _drain(batch, jj, p)
            if jj == 0 and not first:
                _prefetch_idx(nxt, 1 - p)
        _wait_idx(nxt, 1 - p)
        _gather(nxt, 0, 1 - p)
        _drain(batch, _CHB - 1, p)

    _wait_idx(w, 0)
    _gather(w, 0, 0, wait_prev=False)
    _half_iter(w, 0, first=True)

    @pl.loop(1, _BSTEPS, step=2)
    def _(jb):
        _half_iter(w + jb * _NW, 1)
        _half_iter(w + (jb + 1) * _NW, 0)

    pltpu.make_async_copy(rv[0], acc.at[db[0].at[0]], ssem[0]).wait()
    pltpu.make_async_copy(rv[1], acc.at[db[0].at[1]], ssem[1]).wait()

    pltpu.sync_copy(hist, cnt_sh.at[idrows], add=True)
    plsc.subcore_barrier()

    @pl.when(s == 0)
    def _():
        pltpu.sync_copy(cnt_sh, cnt_hbm.at[c])

    @pl.when(c == 0)
    def _():
        pltpu.sync_copy(acc.at[pl.ds(row0, _RPT)],
                        out0_hbm.at[pl.ds(row0, _RPT)])

    @pl.when(c == 1)
    def _():
        pltpu.sync_copy(acc.at[pl.ds(row0, _RPT)],
                        out1_hbm.at[pl.ds(row0, _RPT)])


def _dot_t(a, w):
    return lax.dot_general(a, w, (((1,), (1,)), ((), ())),
                           preferred_element_type=jnp.float32)


def _mean_conv(pa_ref, pb_ref, ca_ref, cb_ref, h_ref, wl_ref, wr_ref, b_ref):
    ic = 1.0 / jnp.maximum(ca_ref[0] + cb_ref[0], 1.0)
    ri = lax.broadcasted_iota(jnp.int32, (_B, 1), 0)
    sel_row = (lax.broadcasted_iota(jnp.int32, (_B, _SB), 1)
               == ri // _D).astype(jnp.float32)
    sel_lane = (lax.broadcasted_iota(jnp.int32, (_B, _D), 1)
                == ri % _D).astype(jnp.float32)
    t = lax.dot_general(sel_row, ic, (((1,), (0,)), ((), ())),
                        preferred_element_type=jnp.float32)
    inv_col = jnp.sum(t * sel_lane, axis=1, keepdims=True)
    mean = (pa_ref[...] + pb_ref[...]) * inv_col
    y = _dot_t(mean, wl_ref[...]) + _dot_t(h_ref[...], wr_ref[...]) \
        + b_ref[...]
    return jnp.maximum(y, 0.0)


def _layer1_body(pa_ref, pb_ref, ca_ref, cb_ref, h_ref, wl_ref, wr_ref,
                 b_ref, out_ref):
    out_ref[...] = _mean_conv(pa_ref, pb_ref, ca_ref, cb_ref, h_ref,
                              wl_ref, wr_ref, b_ref)


def _layer2fc_body(pa_ref, pb_ref, ca_ref, cb_ref, h_ref, wl_ref, wr_ref,
                   b_ref, gamma_ref, beta_ref, fcw_ref, fcb_ref,
                   out_ref, h2buf, accum):
    ph = pl.program_id(0)
    i = pl.program_id(1)

    @pl.when(ph == 0)
    def _():
        h2 = _mean_conv(pa_ref, pb_ref, ca_ref, cb_ref, h_ref, wl_ref,
                        wr_ref, b_ref)
        h2buf[pl.ds(i * _B, _B), :] = h2

        rows = i * _B + lax.broadcasted_iota(jnp.int32, (_B, 1), 0)
        h2m = jnp.where(rows < _N, h2, 0.0)

        @pl.when(i == 0)
        def _():
            accum[...] = jnp.zeros_like(accum)

        accum[0:1, :] += jnp.sum(h2m, axis=0, keepdims=True)
        accum[1:2, :] += jnp.sum(h2m * h2m, axis=0, keepdims=True)

    @pl.when(ph == 1)
    def _():
        inv_n = 1.0 / _N
        mu = accum[0:1, :] * inv_n
        var = accum[1:2, :] * inv_n - mu * mu
        a = gamma_ref[...] * lax.rsqrt(var + _EPS)
        c0 = beta_ref[...] - mu * a
        h2 = h2buf[pl.ds(i * _B, _B), :]
        out_ref[...] = _dot_t(h2 * a + c0, fcw_ref[...]) + fcb_ref[...]


def _full(shape):
    return pl.BlockSpec(shape, lambda i: tuple(0 for _ in shape))


def _sage_layer1(part_a, part_b, cnt3, h, Wl, Wr, b):
    nblk = _N_PAD // _B
    in_specs = [
        pl.BlockSpec((_B, _D), lambda i: (i, 0)),
        pl.BlockSpec((_B, _D), lambda i: (i, 0)),
        pl.BlockSpec((1, _SB, _D), lambda i: (0, i, 0)),
        pl.BlockSpec((1, _SB, _D), lambda i: (1, i, 0)),
        pl.BlockSpec((_B, _D), lambda i: (i, 0)),
        _full((_D, _D)),
        _full((_D, _D)),
        _full((1, _D)),
    ]
    return pl.pallas_call(
        _layer1_body,
        grid=(nblk,),
        in_specs=in_specs,
        out_specs=pl.BlockSpec((_B, _D), lambda i: (i, 0)),
        out_shape=jax.ShapeDtypeStruct((_N, _D), jnp.float32),
    )(part_a, part_b, cnt3, cnt3, h, Wl, Wr, b.reshape(1, _D))


def kernel(x, edge_index, W1l, b1, W1r, W2l, b2, W2r, gamma, beta, fcW, fcb):
    src = edge_index[0].reshape(_NCHUNK, _K)
    dst = edge_index[1].reshape(_NCHUNK, _K)

    p1a, p1b, cnt3 = _sc_aggregate(x, src, dst)
    h1 = _sage_layer1(p1a, p1b, cnt3, x, W1l, W1r, b1)

    p2a, p2b, _ = _sc_aggregate(h1, src, dst)

    nblk = _N_PAD // _B
    in_specs = [
        pl.BlockSpec((_B, _D), lambda ph, i: (i * (1 - ph), 0)),
        pl.BlockSpec((_B, _D), lambda ph, i: (i * (1 - ph), 0)),
        pl.BlockSpec((1, _SB, _D), lambda ph, i: (0, i * (1 - ph), 0)),
        pl.BlockSpec((1, _SB, _D), lambda ph, i: (1, i * (1 - ph), 0)),
        pl.BlockSpec((_B, _D), lambda ph, i: (i * (1 - ph), 0)),
        pl.BlockSpec((_D, _D), lambda ph, i: (0, 0)),
        pl.BlockSpec((_D, _D), lambda ph, i: (0, 0)),
        pl.BlockSpec((1, _D), lambda ph, i: (0, 0)),
        pl.BlockSpec((1, _D), lambda ph, i: (0, 0)),
        pl.BlockSpec((1, _D), lambda ph, i: (0, 0)),
        pl.BlockSpec((_D, _D), lambda ph, i: (0, 0)),
        pl.BlockSpec((1, _D), lambda ph, i: (0, 0)),
    ]
    out = pl.pallas_call(
        _layer2fc_body,
        grid=(2, nblk),
        in_specs=in_specs,
        out_specs=pl.BlockSpec((_B, _D), lambda ph, i: (i * ph, 0)),
        out_shape=jax.ShapeDtypeStruct((_N, _D), jnp.float32),
        scratch_shapes=[pltpu.VMEM((_N_PAD, _D), jnp.float32),
                        pltpu.VMEM((8, _D), jnp.float32)],
    )(p2a, p2b, cnt3, cnt3, h1, W2l, W2r, b2.reshape(1, _D),
      gamma.reshape(1, _D), beta.reshape(1, _D), fcW, fcb.reshape(1, _D))
    return out


# speedup vs baseline: 1.0274x; 1.0018x over previous
import dataclasses
import functools

import jax
import jax.numpy as jnp
from jax import lax
from jax.experimental import pallas as pl
from jax.experimental.pallas import tpu as pltpu
from jax.experimental.pallas import tpu_sc as plsc

_N = 10000
_E = 320000
_D = 128
_K = 128
_NCHUNK = _E // _K
_NC = 2
_NS = 16
_NW = _NC * _NS
_CHB = 8
_NBATCH = -(-_NCHUNK // _CHB)
_BSTEPS = -(-_NBATCH // _NW)
_N_PAD = 10240
_RPT = _N_PAD // _NS
_ZR = 128
_EPS = 1e-5
_B = 2048
_SB = _B // _D

_mesh = plsc.VectorSubcoreMesh(core_axis_name="c", subcore_axis_name="s",
                               num_cores=_NC)

_sc_params = pltpu.CompilerParams()
if "needs_layout_passes" in pltpu.CompilerParams.__dataclass_fields__:
    _sc_params = dataclasses.replace(_sc_params, needs_layout_passes=False)


@functools.partial(
    pl.kernel,
    compiler_params=_sc_params,
    out_type=[jax.ShapeDtypeStruct((_N_PAD, _D), jnp.float32),
              jax.ShapeDtypeStruct((_N_PAD, _D), jnp.float32),
              jax.ShapeDtypeStruct((_NC, _N_PAD // _D, _D), jnp.float32)],
    mesh=_mesh,
    scratch_types=[
        pltpu.VMEM((_CHB, _K), jnp.int32),
        pltpu.VMEM((_CHB, _K), jnp.int32),
        pltpu.VMEM((_CHB, _K), jnp.int32),
        pltpu.VMEM((_CHB, _K), jnp.int32),
        pltpu.VMEM((_K, _D), jnp.float32),
        pltpu.VMEM((_K, _D), jnp.float32),
        pltpu.VMEM((_N_PAD // _D, _D), jnp.float32),
        pltpu.VMEM((_N_PAD // _D,), jnp.int32),
        pltpu.VMEM_SHARED((_N_PAD, _D), jnp.float32),
        pltpu.VMEM_SHARED((_N_PAD // _D, _D), jnp.float32),
        pltpu.SemaphoreType.DMA,
        pltpu.SemaphoreType.DMA,
        pltpu.SemaphoreType.DMA,
        pltpu.SemaphoreType.DMA,
        pltpu.SemaphoreType.DMA,
        pltpu.SemaphoreType.DMA,
    ],
)
def _sc_aggregate(table_hbm, src_hbm, dst_hbm, out0_hbm, out1_hbm, cnt_hbm,
                  sb0, sb1, db0, db1, r0, r1, hist, idrows,
                  acc, cnt_sh, gsem0, gsem1, isem0, isem1, ssem0, ssem1):
    c = lax.axis_index("c")
    s = lax.axis_index("s")
    w = c * _NS + s
    row0 = pl.multiple_of(s * _RPT, 8)
    ones16 = jnp.ones((16,), jnp.float32)
    _HR = _N_PAD // _D
    sb = (sb0, sb1)
    db = (db0, db1)
    rv = (r0, r1)
    gsem = (gsem0, gsem1)
    isem = (isem0, isem1)
    ssem = (ssem0, ssem1)

    def _idx_copies(batch, p):
        brow = pl.multiple_of(batch * _CHB, 8)
        return (pltpu.make_async_copy(src_hbm.at[pl.ds(brow, _CHB)], sb[p],
                                      isem[p]),
                pltpu.make_async_copy(dst_hbm.at[pl.ds(brow, _CHB)], db[p],
                                      isem[p]))

    def _prefetch_idx(batch, p):
        @pl.when(batch < _NBATCH)
        def _():
            for cp in _idx_copies(batch, p):
                cp.start()

    def _wait_idx(batch, p):
        @pl.when(batch < _NBATCH)
        def _():
            for cp in _idx_copies(batch, p):
                cp.wait()

    _prefetch_idx(w, 0)
    _prefetch_idx(w + _NW, 1)

    @pl.loop(0, _ZR)
    def _(r):
        for col in range(0, _D, 16):
            r1[r, pl.ds(col, 16)] = jnp.zeros((16,), jnp.float32)

    zcp = [pltpu.make_async_copy(r1, acc.at[pl.ds(row0 + b * _ZR, _ZR)],
                                 ssem0) for b in range(_RPT // _ZR)]
    for cp in zcp:
        cp.start()
    ccp = pltpu.make_async_copy(r1.at[pl.ds(0, _HR)], cnt_sh, ssem1)

    @pl.when(s == 0)
    def _():
        ccp.start()

    @pl.loop(0, _HR)
    def _(r):
        for col in range(0, _D, 16):
            hist[r, pl.ds(col, 16)] = jnp.zeros((16,), jnp.float32)

    for g in range(_HR // 16):
        idrows[pl.ds(g * 16, 16)] = (lax.iota(jnp.int32, 16)
                                     + jnp.int32(g * 16))

    for cp in zcp:
        cp.wait()

    @pl.when(s == 0)
    def _():
        ccp.wait()
    plsc.subcore_barrier()

    def _gather(batch, jj, p, wait_prev=True):
        @pl.when(batch * _CHB + jj < _NCHUNK)
        def _():
            if wait_prev:
                pltpu.make_async_copy(rv[jj % 2], acc.at[db[p].at[jj]],
                                      ssem[jj % 2]).wait()
            pltpu.async_copy(table_hbm.at[sb[p].at[jj]], rv[jj % 2],
                             gsem[jj % 2])

    def _drain(batch, jj, p):
        @pl.when(batch * _CHB + jj < _NCHUNK)
        def _():
            for g in range(_K // 16):
                dvec = db[p][jj, pl.ds(g * 16, 16)]
                plsc.addupdate_scatter(
                    hist,
                    [lax.shift_right_logical(dvec, 7),
                     lax.bitwise_and(dvec, jnp.int32(_D - 1))],
                    ones16)
            pltpu.make_async_copy(table_hbm.at[sb[p].at[jj]], rv[jj % 2],
                                  gsem[jj % 2]).wait()
            pltpu.async_copy(rv[jj % 2], acc.at[db[p].at[jj]], ssem[jj % 2],
                             add=True)

    def _half_iter(batch, p, first=False):
        nxt = batch + _NW
        for jj in range(_CHB - 1):
            _gather(batch, jj + 1, p, wait_prev=(not first) or jj + 1 >= 2)
            _drain(batch, jj, p)
            if jj == 0 and not first:
                _prefetch_idx(nxt, 1 - p)
        _wait_idx(nxt, 1 - p)
        _gather(nxt, 0, 1 - p)
        _drain(batch, _CHB - 1, p)

    _wait_idx(w, 0)
    _gather(w, 0, 0, wait_prev=False)
    _half_iter(w, 0, first=True)

    @pl.loop(1, _BSTEPS, step=2)
    def _(jb):
        _half_iter(w + jb * _NW, 1)
        _half_iter(w + (jb + 1) * _NW, 0)

    pltpu.make_async_copy(rv[0], acc.at[db[0].at[0]], ssem[0]).wait()
    pltpu.make_async_copy(rv[1], acc.at[db[0].at[1]], ssem[1]).wait()

    pltpu.sync_copy(hist, cnt_sh.at[idrows], add=True)
    plsc.subcore_barrier()

    @pl.when(s == 0)
    def _():
        pltpu.sync_copy(cnt_sh, cnt_hbm.at[c])

    @pl.when(c == 0)
    def _():
        pltpu.sync_copy(acc.at[pl.ds(row0, _RPT)],
                        out0_hbm.at[pl.ds(row0, _RPT)])

    @pl.when(c == 1)
    def _():
        pltpu.sync_copy(acc.at[pl.ds(row0, _RPT)],
                        out1_hbm.at[pl.ds(row0, _RPT)])


def _dot_t(a, w):
    return lax.dot_general(a, w, (((1,), (1,)), ((), ())),
                           preferred_element_type=jnp.float32)


def _mean_conv(pa_ref, pb_ref, ca_ref, cb_ref, h_ref, wl_ref, wr_ref, b_ref):
    ic = 1.0 / jnp.maximum(ca_ref[0] + cb_ref[0], 1.0)
    ri = lax.broadcasted_iota(jnp.int32, (_B, 1), 0)
    sel_row = (lax.broadcasted_iota(jnp.int32, (_B, _SB), 1)
               == ri // _D).astype(jnp.float32)
    sel_lane = (lax.broadcasted_iota(jnp.int32, (_B, _D), 1)
                == ri % _D).astype(jnp.float32)
    t = lax.dot_general(sel_row, ic, (((1,), (0,)), ((), ())),
                        preferred_element_type=jnp.float32)
    inv_col = jnp.sum(t * sel_lane, axis=1, keepdims=True)
    mean = (pa_ref[...] + pb_ref[...]) * inv_col
    y = _dot_t(mean, wl_ref[...]) + _dot_t(h_ref[...], wr_ref[...]) \
        + b_ref[...]
    return jnp.maximum(y, 0.0)


def _layer1_body(pa_ref, pb_ref, ca_ref, cb_ref, h_ref, wl_ref, wr_ref,
                 b_ref, out_ref):
    out_ref[...] = _mean_conv(pa_ref, pb_ref, ca_ref, cb_ref, h_ref,
                              wl_ref, wr_ref, b_ref)


def _layer2fc_body(pa_ref, pb_ref, ca_ref, cb_ref, h_ref, wl_ref, wr_ref,
                   b_ref, gamma_ref, beta_ref, fcw_ref, fcb_ref,
                   out_ref, h2buf, accum):
    ph = pl.program_id(0)
    i = pl.program_id(1)

    @pl.when(ph == 0)
    def _():
        h2 = _mean_conv(pa_ref, pb_ref, ca_ref, cb_ref, h_ref, wl_ref,
                        wr_ref, b_ref)
        h2buf[pl.ds(i * _B, _B), :] = h2

        rows = i * _B + lax.broadcasted_iota(jnp.int32, (_B, 1), 0)
        h2m = jnp.where(rows < _N, h2, 0.0)

        @pl.when(i == 0)
        def _():
            accum[...] = jnp.zeros_like(accum)

        accum[0:1, :] += jnp.sum(h2m, axis=0, keepdims=True)
        accum[1:2, :] += jnp.sum(h2m * h2m, axis=0, keepdims=True)

    @pl.when(ph == 1)
    def _():
        inv_n = 1.0 / _N
        mu = accum[0:1, :] * inv_n
        var = accum[1:2, :] * inv_n - mu * mu
        a = gamma_ref[...] * lax.rsqrt(var + _EPS)
        c0 = beta_ref[...] - mu * a
        h2 = h2buf[pl.ds(i * _B, _B), :]
        out_ref[...] = _dot_t(h2 * a + c0, fcw_ref[...]) + fcb_ref[...]


def _full(shape):
    return pl.BlockSpec(shape, lambda i: tuple(0 for _ in shape))


def _sage_layer1(part_a, part_b, cnt3, h, Wl, Wr, b):
    nblk = _N_PAD // _B
    in_specs = [
        pl.BlockSpec((_B, _D), lambda i: (i, 0)),
        pl.BlockSpec((_B, _D), lambda i: (i, 0)),
        pl.BlockSpec((1, _SB, _D), lambda i: (0, i, 0)),
        pl.BlockSpec((1, _SB, _D), lambda i: (1, i, 0)),
        pl.BlockSpec((_B, _D), lambda i: (i, 0)),
        _full((_D, _D)),
        _full((_D, _D)),
        _full((1, _D)),
    ]
    return pl.pallas_call(
        _layer1_body,
        grid=(nblk,),
        in_specs=in_specs,
        out_specs=pl.BlockSpec((_B, _D), lambda i: (i, 0)),
        out_shape=jax.ShapeDtypeStruct((_N, _D), jnp.float32),
    )(part_a, part_b, cnt3, cnt3, h, Wl, Wr, b.reshape(1, _D))


def kernel(x, edge_index, W1l, b1, W1r, W2l, b2, W2r, gamma, beta, fcW, fcb):
    src = edge_index[0].reshape(_NCHUNK, _K)
    dst = edge_index[1].reshape(_NCHUNK, _K)

    p1a, p1b, cnt3 = _sc_aggregate(x, src, dst)
    h1 = _sage_layer1(p1a, p1b, cnt3, x, W1l, W1r, b1)

    p2a, p2b, _ = _sc_aggregate(h1, src, dst)

    nblk = _N_PAD // _B
    in_specs = [
        pl.BlockSpec((_B, _D), lambda ph, i: (i * (1 - ph), 0)),
        pl.BlockSpec((_B, _D), lambda ph, i: (i * (1 - ph), 0)),
        pl.BlockSpec((1, _SB, _D), lambda ph, i: (0, i * (1 - ph), 0)),
        pl.BlockSpec((1, _SB, _D), lambda ph, i: (1, i * (1 - ph), 0)),
        pl.BlockSpec((_B, _D), lambda ph, i: (i * (1 - ph), 0)),
        pl.BlockSpec((_D, _D), lambda ph, i: (0, 0)),
        pl.BlockSpec((_D, _D), lambda ph, i: (0, 0)),
        pl.BlockSpec((1, _D), lambda ph, i: (0, 0)),
        pl.BlockSpec((1, _D), lambda ph, i: (0, 0)),
        pl.BlockSpec((1, _D), lambda ph, i: (0, 0)),
        pl.BlockSpec((_D, _D), lambda ph, i: (0, 0)),
        pl.BlockSpec((1, _D), lambda ph, i: (0, 0)),
    ]
    out = pl.pallas_call(
        _layer2fc_body,
        grid=(2, nblk),
        in_specs=in_specs,
        out_specs=pl.BlockSpec((_B, _D), lambda ph, i: (i * ph, 0)),
        out_shape=jax.ShapeDtypeStruct((_N, _D), jnp.float32),
        scratch_shapes=[pltpu.VMEM((_N_PAD, _D), jnp.float32),
                        pltpu.VMEM((8, _D), jnp.float32)],
    )(p2a, p2b, cnt3, cnt3, h1, W2l, W2r, b2.reshape(1, _D),
      gamma.reshape(1, _D), beta.reshape(1, _D), fcW, fcb.reshape(1, _D))
    return out


# speedup vs baseline: 1.0310x; 1.0036x over previous
import dataclasses
import functools

import jax
import jax.numpy as jnp
from jax import lax
from jax.experimental import pallas as pl
from jax.experimental.pallas import tpu as pltpu
from jax.experimental.pallas import tpu_sc as plsc

_N = 10000
_E = 320000
_D = 128
_K = 128
_NCHUNK = _E // _K
_NC = 2
_NS = 16
_NW = _NC * _NS
_CHB = 8
_NBATCH = -(-_NCHUNK // _CHB)
_BSTEPS = -(-_NBATCH // _NW)
_N_PAD = 10240
_RPT = _N_PAD // _NS
_ZR = 128
_EPS = 1e-5
_B = 2048
_SB = _B // _D

_mesh = plsc.VectorSubcoreMesh(core_axis_name="c", subcore_axis_name="s",
                               num_cores=_NC)

_sc_params = pltpu.CompilerParams()
if "needs_layout_passes" in pltpu.CompilerParams.__dataclass_fields__:
    _sc_params = dataclasses.replace(_sc_params, needs_layout_passes=False)


@functools.partial(
    pl.kernel,
    compiler_params=_sc_params,
    out_type=[jax.ShapeDtypeStruct((_N_PAD, _D), jnp.float32),
              jax.ShapeDtypeStruct((_N_PAD, _D), jnp.float32),
              jax.ShapeDtypeStruct((_NC, _N_PAD // _D, _D), jnp.float32)],
    mesh=_mesh,
    scratch_types=[
        pltpu.VMEM((_CHB, _K), jnp.int32),
        pltpu.VMEM((_CHB, _K), jnp.int32),
        pltpu.VMEM((_CHB, _K), jnp.int32),
        pltpu.VMEM((_CHB, _K), jnp.int32),
        pltpu.VMEM((_K, _D), jnp.float32),
        pltpu.VMEM((_K, _D), jnp.float32),
        pltpu.VMEM((_N_PAD // _D, _D), jnp.float32),
        pltpu.VMEM((_N_PAD // _D,), jnp.int32),
        pltpu.VMEM_SHARED((_N_PAD, _D), jnp.float32),
        pltpu.VMEM_SHARED((_N_PAD // _D, _D), jnp.float32),
        pltpu.SemaphoreType.DMA,
        pltpu.SemaphoreType.DMA,
        pltpu.SemaphoreType.DMA,
        pltpu.SemaphoreType.DMA,
        pltpu.SemaphoreType.DMA,
        pltpu.SemaphoreType.DMA,
    ],
)
def _sc_aggregate(table_hbm, src_hbm, dst_hbm, out0_hbm, out1_hbm, cnt_hbm,
                  sb0, sb1, db0, db1, r0, r1, hist, idrows,
                  acc, cnt_sh, gsem0, gsem1, isem0, isem1, ssem0, ssem1):
    c = lax.axis_index("c")
    s = lax.axis_index("s")
    w = c * _NS + s
    row0 = pl.multiple_of(s * _RPT, 8)
    ones16 = jnp.ones((16,), jnp.float32)
    _HR = _N_PAD // _D
    sb = (sb0, sb1)
    db = (db0, db1)
    rv = (r0, r1)
    gsem = (gsem0, gsem1)
    isem = (isem0, isem1)
    ssem = (ssem0, ssem1)

    def _idx_copies(batch, p):
        brow = pl.multiple_of(batch * _CHB, 8)
        return (pltpu.make_async_copy(src_hbm.at[pl.ds(brow, _CHB)],
                                      sb[p], isem[p]),
                pltpu.make_async_copy(dst_hbm.at[pl.ds(brow, _CHB)], db[p],
                                      isem[p]))

    def _prefetch_idx(batch, p):
        @pl.when(batch < _NBATCH)
        def _():
            for cp in _idx_copies(batch, p):
                cp.start()

    def _wait_idx(batch, p):
        @pl.when(batch < _NBATCH)
        def _():
            for cp in _idx_copies(batch, p):
                cp.wait()

    _prefetch_idx(w, 0)
    _prefetch_idx(w + _NW, 1)

    @pl.loop(0, _ZR)
    def _(r):
        for col in range(0, _D, 16):
            r1[r, pl.ds(col, 16)] = jnp.zeros((16,), jnp.float32)

    zcp = [pltpu.make_async_copy(r1, acc.at[pl.ds(row0 + b * _ZR, _ZR)],
                                 ssem0) for b in range(_RPT // _ZR)]
    for cp in zcp:
        cp.start()
    ccp = pltpu.make_async_copy(r1.at[pl.ds(0, _HR)], cnt_sh, ssem1)

    @pl.when(s == 0)
    def _():
        ccp.start()

    @pl.loop(0, _HR)
    def _(r):
        for col in range(0, _D, 16):
            hist[r, pl.ds(col, 16)] = jnp.zeros((16,), jnp.float32)

    for g in range(_HR // 16):
        idrows[pl.ds(g * 16, 16)] = (lax.iota(jnp.int32, 16)
                                     + jnp.int32(g * 16))

    for cp in zcp:
        cp.wait()

    @pl.when(s == 0)
    def _():
        ccp.wait()
    plsc.subcore_barrier()

    def _gather(batch, jj, p, wait_prev=True):
        @pl.when(batch * _CHB + jj < _NCHUNK)
        def _():
            if wait_prev:
                pltpu.make_async_copy(rv[jj % 2], acc.at[db[p].at[jj]],
                                      ssem[jj % 2]).wait()
            half = _K // 2
            pltpu.async_copy(table_hbm.at[sb[p].at[jj, pl.ds(0, half)]],
                             rv[jj % 2].at[pl.ds(0, half)], gsem[jj % 2])
            pltpu.async_copy(table_hbm.at[sb[p].at[jj, pl.ds(half, half)]],
                             rv[jj % 2].at[pl.ds(half, half)], gsem[jj % 2])

    def _drain(batch, jj, p):
        @pl.when(batch * _CHB + jj < _NCHUNK)
        def _():
            for g in range(_K // 16):
                dvec = db[p][jj, pl.ds(g * 16, 16)]
                plsc.addupdate_scatter(
                    hist,
                    [lax.shift_right_logical(dvec, 7),
                     lax.bitwise_and(dvec, jnp.int32(_D - 1))],
                    ones16)
            half = _K // 2
            pltpu.make_async_copy(table_hbm.at[sb[p].at[jj, pl.ds(0, half)]],
                                  rv[jj % 2].at[pl.ds(0, half)],
                                  gsem[jj % 2]).wait()
            pltpu.make_async_copy(
                table_hbm.at[sb[p].at[jj, pl.ds(half, half)]],
                rv[jj % 2].at[pl.ds(half, half)],
                gsem[jj % 2]).wait()
            pltpu.async_copy(rv[jj % 2], acc.at[db[p].at[jj]], ssem[jj % 2],
                             add=True)

    def _half_iter(batch, p, first=False):
        nxt = batch + _NW
        for jj in range(_CHB - 1):
            _gather(batch, jj + 1, p, wait_prev=(not first) or jj + 1 >= 2)
            _drain(batch, jj, p)
            if jj == 0 and not first:
                _prefetch_idx(nxt, 1 - p)
        _wait_idx(nxt, 1 - p)
        _gather(nxt, 0, 1 - p)
        _drain(batch, _CHB - 1, p)

    _wait_idx(w, 0)
    _gather(w, 0, 0, wait_prev=False)
    _half_iter(w, 0, first=True)

    @pl.loop(1, _BSTEPS, step=2)
    def _(jb):
        _half_iter(w + jb * _NW, 1)
        _half_iter(w + (jb + 1) * _NW, 0)

    pltpu.make_async_copy(rv[0], acc.at[db[0].at[0]], ssem[0]).wait()
    pltpu.make_async_copy(rv[1], acc.at[db[0].at[1]], ssem[1]).wait()

    pltpu.sync_copy(hist, cnt_sh.at[idrows], add=True)
    plsc.subcore_barrier()

    @pl.when(s == 0)
    def _():
        pltpu.sync_copy(cnt_sh, cnt_hbm.at[c])

    @pl.when(c == 0)
    def _():
        pltpu.sync_copy(acc.at[pl.ds(row0, _RPT)],
                        out0_hbm.at[pl.ds(row0, _RPT)])

    @pl.when(c == 1)
    def _():
        pltpu.sync_copy(acc.at[pl.ds(row0, _RPT)],
                        out1_hbm.at[pl.ds(row0, _RPT)])


def _dot_t(a, w):
    return lax.dot_general(a, w, (((1,), (1,)), ((), ())),
                           preferred_element_type=jnp.float32)


def _mean_conv(pa_ref, pb_ref, ca_ref, cb_ref, h_ref, wl_ref, wr_ref, b_ref):
    ic = 1.0 / jnp.maximum(ca_ref[0] + cb_ref[0], 1.0)
    ri = lax.broadcasted_iota(jnp.int32, (_B, 1), 0)
    sel_row = (lax.broadcasted_iota(jnp.int32, (_B, _SB), 1)
               == ri // _D).astype(jnp.float32)
    sel_lane = (lax.broadcasted_iota(jnp.int32, (_B, _D), 1)
                == ri % _D).astype(jnp.float32)
    t = lax.dot_general(sel_row, ic, (((1,), (0,)), ((), ())),
                        preferred_element_type=jnp.float32)
    inv_col = jnp.sum(t * sel_lane, axis=1, keepdims=True)
    mean = (pa_ref[...] + pb_ref[...]) * inv_col
    y = _dot_t(mean, wl_ref[...]) + _dot_t(h_ref[...], wr_ref[...]) \
        + b_ref[...]
    return jnp.maximum(y, 0.0)


def _layer1_body(pa_ref, pb_ref, ca_ref, cb_ref, h_ref, wl_ref, wr_ref,
                 b_ref, out_ref):
    out_ref[...] = _mean_conv(pa_ref, pb_ref, ca_ref, cb_ref, h_ref,
                              wl_ref, wr_ref, b_ref)


def _layer2fc_body(pa_ref, pb_ref, ca_ref, cb_ref, h_ref, wl_ref, wr_ref,
                   b_ref, gamma_ref, beta_ref, fcw_ref, fcb_ref,
                   out_ref, h2buf, accum):
    ph = pl.program_id(0)
    i = pl.program_id(1)

    @pl.when(ph == 0)
    def _():
        h2 = _mean_conv(pa_ref, pb_ref, ca_ref, cb_ref, h_ref, wl_ref,
                        wr_ref, b_ref)
        h2buf[pl.ds(i * _B, _B), :] = h2

        rows = i * _B + lax.broadcasted_iota(jnp.int32, (_B, 1), 0)
        h2m = jnp.where(rows < _N, h2, 0.0)

        @pl.when(i == 0)
        def _():
            accum[...] = jnp.zeros_like(accum)

        accum[0:1, :] += jnp.sum(h2m, axis=0, keepdims=True)
        accum[1:2, :] += jnp.sum(h2m * h2m, axis=0, keepdims=True)

    @pl.when(ph == 1)
    def _():
        inv_n = 1.0 / _N
        mu = accum[0:1, :] * inv_n
        var = accum[1:2, :] * inv_n - mu * mu
        a = gamma_ref[...] * lax.rsqrt(var + _EPS)
        c0 = beta_ref[...] - mu * a
        h2 = h2buf[pl.ds(i * _B, _B), :]
        out_ref[...] = _dot_t(h2 * a + c0, fcw_ref[...]) + fcb_ref[...]


def _full(shape):
    return pl.BlockSpec(shape, lambda i: tuple(0 for _ in shape))


def _sage_layer1(part_a, part_b, cnt3, h, Wl, Wr, b):
    nblk = _N_PAD // _B
    in_specs = [
        pl.BlockSpec((_B, _D), lambda i: (i, 0)),
        pl.BlockSpec((_B, _D), lambda i: (i, 0)),
        pl.BlockSpec((1, _SB, _D), lambda i: (0, i, 0)),
        pl.BlockSpec((1, _SB, _D), lambda i: (1, i, 0)),
        pl.BlockSpec((_B, _D), lambda i: (i, 0)),
        _full((_D, _D)),
        _full((_D, _D)),
        _full((1, _D)),
    ]
    return pl.pallas_call(
        _layer1_body,
        grid=(nblk,),
        in_specs=in_specs,
        out_specs=pl.BlockSpec((_B, _D), lambda i: (i, 0)),
        out_shape=jax.ShapeDtypeStruct((_N, _D), jnp.float32),
    )(part_a, part_b, cnt3, cnt3, h, Wl, Wr, b.reshape(1, _D))


def kernel(x, edge_index, W1l, b1, W1r, W2l, b2, W2r, gamma, beta, fcW, fcb):
    src = edge_index[0].reshape(_NCHUNK, _K)
    dst = edge_index[1].reshape(_NCHUNK, _K)

    p1a, p1b, cnt3 = _sc_aggregate(x, src, dst)
    h1 = _sage_layer1(p1a, p1b, cnt3, x, W1l, W1r, b1)

    p2a, p2b, _ = _sc_aggregate(h1, src, dst)

    nblk = _N_PAD // _B
    in_specs = [
        pl.BlockSpec((_B, _D), lambda ph, i: (i * (1 - ph), 0)),
        pl.BlockSpec((_B, _D), lambda ph, i: (i * (1 - ph), 0)),
        pl.BlockSpec((1, _SB, _D), lambda ph, i: (0, i * (1 - ph), 0)),
        pl.BlockSpec((1, _SB, _D), lambda ph, i: (1, i * (1 - ph), 0)),
        pl.BlockSpec((_B, _D), lambda ph, i: (i * (1 - ph), 0)),
        pl.BlockSpec((_D, _D), lambda ph, i: (0, 0)),
        pl.BlockSpec((_D, _D), lambda ph, i: (0, 0)),
        pl.BlockSpec((1, _D), lambda ph, i: (0, 0)),
        pl.BlockSpec((1, _D), lambda ph, i: (0, 0)),
        pl.BlockSpec((1, _D), lambda ph, i: (0, 0)),
        pl.BlockSpec((_D, _D), lambda ph, i: (0, 0)),
        pl.BlockSpec((1, _D), lambda ph, i: (0, 0)),
    ]
    out = pl.pallas_call(
        _layer2fc_body,
        grid=(2, nblk),
        in_specs=in_specs,
        out_specs=pl.BlockSpec((_B, _D), lambda ph, i: (i * ph, 0)),
        out_shape=jax.ShapeDtypeStruct((_N, _D), jnp.float32),
        scratch_shapes=[pltpu.VMEM((_N_PAD, _D), jnp.float32),
                        pltpu.VMEM((8, _D), jnp.float32)],
    )(p2a, p2b, cnt3, cnt3, h1, W2l, W2r, b2.reshape(1, _D),
      gamma.reshape(1, _D), beta.reshape(1, _D), fcW, fcb.reshape(1, _D))
    return out
